# Initial kernel scaffold; baseline (speedup 1.0000x reference)
import jax
import jax.numpy as jnp
from jax.experimental import pallas as pl


def kernel(x, edge_index, edge_attr, batch, W_enc, b_enc, W1_l, b1_l, W1_r, W2_l, b2_l, W2_r, W_m1, b_m1, W_m2, b_m2):
    raise NotImplementedError("write your pallas kernel here")


# speedup vs baseline: 4.7704x; 4.7704x over previous
import functools

import jax
import jax.numpy as jnp
from jax import lax
from jax.experimental import pallas as pl
from jax.experimental.pallas import tpu as pltpu
import jax.experimental.pallas.tpu_sc as plsc

N_CORES = 2
N_SUB = 16
NW = N_CORES * N_SUB
CHUNK = 128
D = 128
CNT_W = 16
BLK = 1024


N_COL = 4
CW = D // N_COL


def _sc_scatter_body(with_cnt, *args):
    h_hbms = args[:N_COL]
    src_hbm = args[N_COL]
    dst_hbm = args[N_COL + 1]
    rest = args[N_COL + 2:]
    if with_cnt:
        parts = rest[:N_COL]
        cnt_hbm = rest[N_COL]
        accs = rest[N_COL + 1:2 * N_COL + 1]
        cnt_sh = rest[2 * N_COL + 1]
        rows_qs = rest[2 * N_COL + 2:3 * N_COL + 2]
        src_v, dst_v, ones_v, sem = rest[3 * N_COL + 2:]
    else:
        parts = rest[:N_COL]
        accs = rest[N_COL:2 * N_COL]
        rows_qs = rest[2 * N_COL:3 * N_COL]
        src_v, dst_v, sem = rest[3 * N_COL:]

    c = lax.axis_index("c")
    s = lax.axis_index("s")
    wid = c * N_SUB + s
    n_chunks = src_hbm.shape[0] // (NW * CHUNK)
    rows = accs[0].shape[0]
    rows_per_tile = rows // N_SUB
    base = s * rows_per_tile
    edge_base = wid * n_chunks * CHUNK

    def fill_zero(i, _):
        for q in range(N_COL):
            for j in range(CW // 16):
                rows_qs[q][i, pl.ds(j * 16, 16)] = jnp.zeros((16,), jnp.float32)
        if with_cnt:
            ones_v[i, pl.ds(0, 16)] = jnp.zeros((16,), jnp.float32)
        return 0
    lax.fori_loop(0, CHUNK, fill_zero, 0)

    @pl.loop(0, rows_per_tile // CHUNK)
    def zero_slice(k):
        off = base + k * CHUNK
        for q in range(N_COL):
            pltpu.sync_copy(rows_qs[q], accs[q].at[pl.ds(off, CHUNK)])
        if with_cnt:
            pltpu.sync_copy(ones_v, cnt_sh.at[pl.ds(off, CHUNK)])
    if with_cnt:
        def fill_ones(i, _):
            ones_v[i, pl.ds(0, 16)] = jnp.ones((16,), jnp.float32)
            return 0
        lax.fori_loop(0, CHUNK, fill_ones, 0)
    plsc.subcore_barrier()

    @pl.loop(0, n_chunks)
    def chunk_step(j):
        off = edge_base + j * CHUNK
        pltpu.sync_copy(src_hbm.at[pl.ds(off, CHUNK)], src_v)
        pltpu.sync_copy(dst_hbm.at[pl.ds(off, CHUNK)], dst_v)
        gathers = [pltpu.async_copy(h_hbms[q].at[src_v], rows_qs[q], sem)
                   for q in range(N_COL)]
        for g in gathers:
            g.wait()
        for q in range(N_COL):
            pltpu.sync_copy(rows_qs[q], accs[q].at[dst_v], add=True)
        if with_cnt:
            pltpu.sync_copy(ones_v, cnt_sh.at[dst_v], add=True)
    plsc.subcore_barrier()

    @pl.loop(0, rows_per_tile // CHUNK)
    def flush(k):
        off = base + k * CHUNK
        for q in range(N_COL):
            pltpu.sync_copy(accs[q].at[pl.ds(off, CHUNK)], rows_qs[q])
            pltpu.sync_copy(rows_qs[q], parts[q].at[pl.ds(c * rows + off, CHUNK)])
        if with_cnt:
            pltpu.sync_copy(cnt_sh.at[pl.ds(off, CHUNK)], ones_v)
            pltpu.sync_copy(ones_v, cnt_hbm.at[pl.ds(c * rows + off, CHUNK)])


def _make_sc_scatter(rows, n_chunks, with_cnt):
    mesh = plsc.VectorSubcoreMesh(core_axis_name="c", subcore_axis_name="s")
    out_type = [jax.ShapeDtypeStruct((N_CORES * rows, CW), jnp.float32)
                for _ in range(N_COL)]
    if with_cnt:
        out_type.append(jax.ShapeDtypeStruct((N_CORES * rows, CNT_W), jnp.float32))
    scratch = [pltpu.VMEM_SHARED((rows, CW), jnp.float32) for _ in range(N_COL)]
    if with_cnt:
        scratch.append(pltpu.VMEM_SHARED((rows, CNT_W), jnp.float32))
    scratch += [pltpu.VMEM((CHUNK, CW), jnp.float32) for _ in range(N_COL)]
    scratch += [
        pltpu.VMEM((CHUNK,), jnp.int32),
        pltpu.VMEM((CHUNK,), jnp.int32),
    ]
    if with_cnt:
        scratch.append(pltpu.VMEM((CHUNK, CNT_W), jnp.float32))
    scratch.append(pltpu.SemaphoreType.DMA)
    return pl.kernel(
        functools.partial(_sc_scatter_body, with_cnt),
        out_type=tuple(out_type),
        mesh=mesh,
        scratch_types=tuple(scratch),
        compiler_params=pltpu.CompilerParams(use_tc_tiling_on_sc=False),
    )


def _split_store(o_refs, val):
    for q in range(N_COL):
        o_refs[q][...] = val[:, q * CW:(q + 1) * CW]


def _enc_body(x_ref, w_ref, b_ref, *o_refs):
    _split_store(o_refs, jnp.dot(x_ref[...], w_ref[...],
                                 preferred_element_type=jnp.float32) + b_ref[...])


def _mean_h(p_refs, cnt_ref, h_refs):
    agg = jnp.concatenate([r[...][0] + r[...][1] for r in p_refs], axis=1)
    deg = cnt_ref[0, :, :1] + cnt_ref[1, :, :1]
    mean = agg * (1.0 / jnp.maximum(deg, 1.0))
    h = jnp.concatenate([r[...] for r in h_refs], axis=1)
    return mean, h


def _combine_body(*refs):
    p_refs, cnt_ref, h_refs = refs[:N_COL], refs[N_COL], refs[N_COL + 1:2 * N_COL + 1]
    wl_ref, wr_ref, b_ref = refs[2 * N_COL + 1:2 * N_COL + 4]
    o_refs = refs[2 * N_COL + 4:]
    mean, h = _mean_h(p_refs, cnt_ref, h_refs)
    o = (jnp.dot(mean, wl_ref[...], preferred_element_type=jnp.float32)
         + jnp.dot(h, wr_ref[...], preferred_element_type=jnp.float32)
         + b_ref[...])
    _split_store(o_refs, jnp.maximum(o, 0.0))


def _combine_head_body(*refs):
    p_refs, cnt_ref, h_refs = refs[:N_COL], refs[N_COL], refs[N_COL + 1:2 * N_COL + 1]
    (wl_ref, wr_ref, b_ref, wm1_ref, bm1_ref, wm2_ref, bm2_ref,
     o_ref) = refs[2 * N_COL + 1:]
    mean, h = _mean_h(p_refs, cnt_ref, h_refs)
    h2 = (jnp.dot(mean, wl_ref[...], preferred_element_type=jnp.float32)
          + jnp.dot(h, wr_ref[...], preferred_element_type=jnp.float32)
          + b_ref[...])
    h2 = jnp.maximum(h2, 0.0)
    r = jnp.maximum(jnp.dot(h2, wm1_ref[...],
                            preferred_element_type=jnp.float32) + bm1_ref[...],
                    0.0)
    logit = jnp.sum(r * wm2_ref[...], axis=1) + bm2_ref[0, 0]
    o_ref[...] = (1.0 / (1.0 + jnp.exp(-logit)))[None, :]


def _full_spec(shape):
    return pl.BlockSpec(shape, lambda i: tuple(0 for _ in shape))


def kernel(x, edge_index, edge_attr, batch, W_enc, b_enc, W1_l, b1_l, W1_r,
           W2_l, b2_l, W2_r, W_m1, b_m1, W_m2, b_m2):
    n = x.shape[0]
    e = edge_index.shape[1]
    rows = pl.cdiv(n + 1, BLK) * BLK
    n_chunks = pl.cdiv(e, NW * CHUNK)
    e_pad = NW * CHUNK * n_chunks

    src = edge_index[0].astype(jnp.int32)
    dst = edge_index[1].astype(jnp.int32)
    src = jnp.concatenate([src, jnp.zeros((e_pad - e,), jnp.int32)])
    dst = jnp.concatenate([dst, jnp.full((e_pad - e,), n, jnp.int32)])

    x_p = jnp.pad(x, ((0, rows - n), (0, 0)))
    b_enc2 = b_enc.reshape(1, D)
    b1 = b1_l.reshape(1, D)
    b2 = b2_l.reshape(1, D)
    bm1 = b_m1.reshape(1, D)
    wm2 = W_m2.reshape(1, D)
    bm2 = b_m2.reshape(1, 1)

    grid = rows // BLK
    row_spec = pl.BlockSpec((BLK, CW), lambda i: (i, 0))
    part_spec = pl.BlockSpec((N_CORES, BLK, CW), lambda i: (0, i, 0))
    cnt_spec = pl.BlockSpec((N_CORES, BLK, CNT_W), lambda i: (0, i, 0))
    nq_shape = [jax.ShapeDtypeStruct((rows, CW), jnp.float32)] * N_COL
    nq_spec = [row_spec] * N_COL

    h0s = pl.pallas_call(
        _enc_body,
        grid=(grid,),
        in_specs=[pl.BlockSpec((BLK, D), lambda i: (i, 0)),
                  _full_spec((D, D)), _full_spec((1, D))],
        out_specs=nq_spec,
        out_shape=nq_shape,
    )(x_p, W_enc, b_enc2)

    sc1 = _make_sc_scatter(rows, n_chunks, with_cnt=True)
    out1 = sc1(*h0s, src, dst)
    part1 = [p.reshape(N_CORES, rows, CW) for p in out1[:N_COL]]
    cnt = out1[N_COL].reshape(N_CORES, rows, CNT_W)

    h1s = pl.pallas_call(
        _combine_body,
        grid=(grid,),
        in_specs=[part_spec] * N_COL + [cnt_spec] + nq_spec
                 + [_full_spec((D, D)), _full_spec((D, D)), _full_spec((1, D))],
        out_specs=nq_spec,
        out_shape=nq_shape,
    )(*part1, cnt, *h0s, W1_l, W1_r, b1)

    sc2 = _make_sc_scatter(rows, n_chunks, with_cnt=False)
    out2 = sc2(*h1s, src, dst)
    part2 = [p.reshape(N_CORES, rows, CW) for p in out2]

    out2d = pl.pallas_call(
        _combine_head_body,
        grid=(grid,),
        in_specs=[part_spec] * N_COL + [cnt_spec] + nq_spec
                 + [_full_spec((D, D)), _full_spec((D, D)), _full_spec((1, D)),
                    _full_spec((D, D)), _full_spec((1, D)), _full_spec((1, D)),
                    _full_spec((1, 1))],
        out_specs=pl.BlockSpec((1, BLK), lambda i: (0, i)),
        out_shape=jax.ShapeDtypeStruct((1, rows), jnp.float32),
    )(*part2, cnt, *h1s, W2_l, W2_r, b2, W_m1, bm1, wm2, bm2)

    return out2d[0, :n]


# speedup vs baseline: 5.3912x; 1.1301x over previous
import functools

import jax
import jax.numpy as jnp
from jax import lax
from jax.experimental import pallas as pl
from jax.experimental.pallas import tpu as pltpu
import jax.experimental.pallas.tpu_sc as plsc

N_CORES = 2
N_SUB = 16
NW = N_CORES * N_SUB
CHUNK = 128
D = 128
CNT_W = 16
BLK = 1024


N_COL = 4
CW = D // N_COL


def _sc_scatter_body(with_cnt, *args):
    h_hbms = args[:N_COL]
    src_hbm = args[N_COL]
    dst_hbm = args[N_COL + 1]
    rest = args[N_COL + 2:]
    if with_cnt:
        parts = rest[:N_COL]
        cnt_hbm = rest[N_COL]
        accs = rest[N_COL + 1:2 * N_COL + 1]
        cnt_sh = rest[2 * N_COL + 1]
        rows_qs = rest[2 * N_COL + 2:3 * N_COL + 2]
        src_v, dst_v, ones_v, sem = rest[3 * N_COL + 2:]
    else:
        parts = rest[:N_COL]
        accs = rest[N_COL:2 * N_COL]
        rows_qs = rest[2 * N_COL:3 * N_COL]
        src_v, dst_v, sem = rest[3 * N_COL:]

    c = lax.axis_index("c")
    s = lax.axis_index("s")
    wid = c * N_SUB + s
    n_chunks = src_hbm.shape[0] // (NW * CHUNK)
    rows = accs[0].shape[0]
    rows_per_tile = rows // N_SUB
    base = s * rows_per_tile
    edge_base = wid * n_chunks * CHUNK

    def fill_zero(i, _):
        for q in range(N_COL):
            for j in range(CW // 16):
                rows_qs[q][i, pl.ds(j * 16, 16)] = jnp.zeros((16,), jnp.float32)
        if with_cnt:
            ones_v[i, pl.ds(0, 16)] = jnp.zeros((16,), jnp.float32)
        return 0
    lax.fori_loop(0, CHUNK, fill_zero, 0)

    @pl.loop(0, rows_per_tile // CHUNK)
    def zero_slice(k):
        off = base + k * CHUNK
        for q in range(N_COL):
            pltpu.sync_copy(rows_qs[q], accs[q].at[pl.ds(off, CHUNK)])
        if with_cnt:
            pltpu.sync_copy(ones_v, cnt_sh.at[pl.ds(off, CHUNK)])
    if with_cnt:
        def fill_ones(i, _):
            ones_v[i, pl.ds(0, 16)] = jnp.ones((16,), jnp.float32)
            return 0
        lax.fori_loop(0, CHUNK, fill_ones, 0)
    plsc.subcore_barrier()

    @pl.loop(0, n_chunks)
    def chunk_step(j):
        off = edge_base + j * CHUNK
        idx = [pltpu.async_copy(src_hbm.at[pl.ds(off, CHUNK)], src_v, sem),
               pltpu.async_copy(dst_hbm.at[pl.ds(off, CHUNK)], dst_v, sem)]
        for i in idx:
            i.wait()
        gathers = [pltpu.async_copy(h_hbms[q].at[src_v], rows_qs[q], sem)
                   for q in range(N_COL)]
        for g in gathers:
            g.wait()
        scat = [pltpu.async_copy(rows_qs[q], accs[q].at[dst_v], sem, add=True)
                for q in range(N_COL)]
        if with_cnt:
            scat.append(pltpu.async_copy(ones_v, cnt_sh.at[dst_v], sem,
                                         add=True))
        for sc in scat:
            sc.wait()
    plsc.subcore_barrier()

    @pl.loop(0, rows_per_tile // CHUNK)
    def flush(k):
        off = base + k * CHUNK
        for q in range(N_COL):
            pltpu.sync_copy(accs[q].at[pl.ds(off, CHUNK)], rows_qs[q])
            pltpu.sync_copy(rows_qs[q], parts[q].at[pl.ds(c * rows + off, CHUNK)])
        if with_cnt:
            pltpu.sync_copy(cnt_sh.at[pl.ds(off, CHUNK)], ones_v)
            pltpu.sync_copy(ones_v, cnt_hbm.at[pl.ds(c * rows + off, CHUNK)])


def _make_sc_scatter(rows, n_chunks, with_cnt):
    mesh = plsc.VectorSubcoreMesh(core_axis_name="c", subcore_axis_name="s")
    out_type = [jax.ShapeDtypeStruct((N_CORES * rows, CW), jnp.float32)
                for _ in range(N_COL)]
    if with_cnt:
        out_type.append(jax.ShapeDtypeStruct((N_CORES * rows, CNT_W), jnp.float32))
    scratch = [pltpu.VMEM_SHARED((rows, CW), jnp.float32) for _ in range(N_COL)]
    if with_cnt:
        scratch.append(pltpu.VMEM_SHARED((rows, CNT_W), jnp.float32))
    scratch += [pltpu.VMEM((CHUNK, CW), jnp.float32) for _ in range(N_COL)]
    scratch += [
        pltpu.VMEM((CHUNK,), jnp.int32),
        pltpu.VMEM((CHUNK,), jnp.int32),
    ]
    if with_cnt:
        scratch.append(pltpu.VMEM((CHUNK, CNT_W), jnp.float32))
    scratch.append(pltpu.SemaphoreType.DMA)
    return pl.kernel(
        functools.partial(_sc_scatter_body, with_cnt),
        out_type=tuple(out_type),
        mesh=mesh,
        scratch_types=tuple(scratch),
        compiler_params=pltpu.CompilerParams(use_tc_tiling_on_sc=False),
    )


def _split_store(o_refs, val):
    for q in range(N_COL):
        o_refs[q][...] = val[:, q * CW:(q + 1) * CW]


def _enc_body(x_ref, w_ref, b_ref, *o_refs):
    _split_store(o_refs, jnp.dot(x_ref[...], w_ref[...],
                                 preferred_element_type=jnp.float32) + b_ref[...])


def _mean_h(p_refs, cnt_ref, h_refs):
    agg = jnp.concatenate([r[...][0] + r[...][1] for r in p_refs], axis=1)
    deg = cnt_ref[0, :, :1] + cnt_ref[1, :, :1]
    mean = agg * (1.0 / jnp.maximum(deg, 1.0))
    h = jnp.concatenate([r[...] for r in h_refs], axis=1)
    return mean, h


def _combine_body(*refs):
    p_refs, cnt_ref, h_refs = refs[:N_COL], refs[N_COL], refs[N_COL + 1:2 * N_COL + 1]
    wl_ref, wr_ref, b_ref = refs[2 * N_COL + 1:2 * N_COL + 4]
    o_refs = refs[2 * N_COL + 4:]
    mean, h = _mean_h(p_refs, cnt_ref, h_refs)
    o = (jnp.dot(mean, wl_ref[...], preferred_element_type=jnp.float32)
         + jnp.dot(h, wr_ref[...], preferred_element_type=jnp.float32)
         + b_ref[...])
    _split_store(o_refs, jnp.maximum(o, 0.0))


def _combine_head_body(*refs):
    p_refs, cnt_ref, h_refs = refs[:N_COL], refs[N_COL], refs[N_COL + 1:2 * N_COL + 1]
    (wl_ref, wr_ref, b_ref, wm1_ref, bm1_ref, wm2_ref, bm2_ref,
     o_ref) = refs[2 * N_COL + 1:]
    mean, h = _mean_h(p_refs, cnt_ref, h_refs)
    h2 = (jnp.dot(mean, wl_ref[...], preferred_element_type=jnp.float32)
          + jnp.dot(h, wr_ref[...], preferred_element_type=jnp.float32)
          + b_ref[...])
    h2 = jnp.maximum(h2, 0.0)
    r = jnp.maximum(jnp.dot(h2, wm1_ref[...],
                            preferred_element_type=jnp.float32) + bm1_ref[...],
                    0.0)
    logit = jnp.sum(r * wm2_ref[...], axis=1) + bm2_ref[0, 0]
    o_ref[...] = (1.0 / (1.0 + jnp.exp(-logit)))[None, :]


def _full_spec(shape):
    return pl.BlockSpec(shape, lambda i: tuple(0 for _ in shape))


def kernel(x, edge_index, edge_attr, batch, W_enc, b_enc, W1_l, b1_l, W1_r,
           W2_l, b2_l, W2_r, W_m1, b_m1, W_m2, b_m2):
    n = x.shape[0]
    e = edge_index.shape[1]
    rows = pl.cdiv(n + 1, BLK) * BLK
    n_chunks = pl.cdiv(e, NW * CHUNK)
    e_pad = NW * CHUNK * n_chunks

    src = edge_index[0].astype(jnp.int32)
    dst = edge_index[1].astype(jnp.int32)
    src = jnp.concatenate([src, jnp.zeros((e_pad - e,), jnp.int32)])
    dst = jnp.concatenate([dst, jnp.full((e_pad - e,), n, jnp.int32)])

    x_p = jnp.pad(x, ((0, rows - n), (0, 0)))
    b_enc2 = b_enc.reshape(1, D)
    b1 = b1_l.reshape(1, D)
    b2 = b2_l.reshape(1, D)
    bm1 = b_m1.reshape(1, D)
    wm2 = W_m2.reshape(1, D)
    bm2 = b_m2.reshape(1, 1)

    grid = rows // BLK
    row_spec = pl.BlockSpec((BLK, CW), lambda i: (i, 0))
    part_spec = pl.BlockSpec((N_CORES, BLK, CW), lambda i: (0, i, 0))
    cnt_spec = pl.BlockSpec((N_CORES, BLK, CNT_W), lambda i: (0, i, 0))
    nq_shape = [jax.ShapeDtypeStruct((rows, CW), jnp.float32)] * N_COL
    nq_spec = [row_spec] * N_COL

    h0s = pl.pallas_call(
        _enc_body,
        grid=(grid,),
        in_specs=[pl.BlockSpec((BLK, D), lambda i: (i, 0)),
                  _full_spec((D, D)), _full_spec((1, D))],
        out_specs=nq_spec,
        out_shape=nq_shape,
    )(x_p, W_enc, b_enc2)

    sc1 = _make_sc_scatter(rows, n_chunks, with_cnt=True)
    out1 = sc1(*h0s, src, dst)
    part1 = [p.reshape(N_CORES, rows, CW) for p in out1[:N_COL]]
    cnt = out1[N_COL].reshape(N_CORES, rows, CNT_W)

    h1s = pl.pallas_call(
        _combine_body,
        grid=(grid,),
        in_specs=[part_spec] * N_COL + [cnt_spec] + nq_spec
                 + [_full_spec((D, D)), _full_spec((D, D)), _full_spec((1, D))],
        out_specs=nq_spec,
        out_shape=nq_shape,
    )(*part1, cnt, *h0s, W1_l, W1_r, b1)

    sc2 = _make_sc_scatter(rows, n_chunks, with_cnt=False)
    out2 = sc2(*h1s, src, dst)
    part2 = [p.reshape(N_CORES, rows, CW) for p in out2]

    out2d = pl.pallas_call(
        _combine_head_body,
        grid=(grid,),
        in_specs=[part_spec] * N_COL + [cnt_spec] + nq_spec
                 + [_full_spec((D, D)), _full_spec((D, D)), _full_spec((1, D)),
                    _full_spec((D, D)), _full_spec((1, D)), _full_spec((1, D)),
                    _full_spec((1, 1))],
        out_specs=pl.BlockSpec((1, BLK), lambda i: (0, i)),
        out_shape=jax.ShapeDtypeStruct((1, rows), jnp.float32),
    )(*part2, cnt, *h1s, W2_l, W2_r, b2, W_m1, bm1, wm2, bm2)

    return out2d[0, :n]


# speedup vs baseline: 5.9624x; 1.1060x over previous
import functools

import jax
import jax.numpy as jnp
from jax import lax
from jax.experimental import pallas as pl
from jax.experimental.pallas import tpu as pltpu
import jax.experimental.pallas.tpu_sc as plsc

N_CORES = 2
N_SUB = 16
NW = N_CORES * N_SUB
CHUNK = 128
D = 128
CNT_W = 16
BLK = 1024


N_COL = 4
CW = D // N_COL


def _sc_scatter_body(with_cnt, *args):
    h_hbms = args[:N_COL]
    src_hbm = args[N_COL]
    dst_hbm = args[N_COL + 1]
    rest = args[N_COL + 2:]
    parts = rest[:N_COL]
    rest = rest[N_COL:]
    if with_cnt:
        cnt_hbm = rest[0]
        rest = rest[1:]
    accs = rest[:N_COL]
    rest = rest[N_COL:]
    if with_cnt:
        cnt_sh = rest[0]
        rest = rest[1:]
    rows2 = (rest[:N_COL], rest[N_COL:2 * N_COL])
    rest = rest[2 * N_COL:]
    src4 = rest[0:4]
    dst4 = rest[4:8]
    rest = rest[8:]
    if with_cnt:
        ones_v = rest[0]
        rest = rest[1:]
    isem, gsem0, gsem1, ssem0, ssem1 = rest
    gsem = (gsem0, gsem1)
    ssem = (ssem0, ssem1)
    rows_qs = rows2[0]

    c = lax.axis_index("c")
    s = lax.axis_index("s")
    wid = c * N_SUB + s
    n_chunks = src_hbm.shape[0] // (NW * CHUNK)
    rows = accs[0].shape[0]
    rows_per_tile = rows // N_SUB
    base = s * rows_per_tile
    edge_base = wid * n_chunks * CHUNK

    def fill_zero(i, _):
        for q in range(N_COL):
            for j in range(CW // 16):
                rows_qs[q][i, pl.ds(j * 16, 16)] = jnp.zeros((16,), jnp.float32)
        if with_cnt:
            ones_v[i, pl.ds(0, 16)] = jnp.zeros((16,), jnp.float32)
        return 0
    lax.fori_loop(0, CHUNK, fill_zero, 0)

    @pl.loop(0, rows_per_tile // CHUNK)
    def zero_slice(k):
        off = base + k * CHUNK
        for q in range(N_COL):
            pltpu.sync_copy(rows_qs[q], accs[q].at[pl.ds(off, CHUNK)])
        if with_cnt:
            pltpu.sync_copy(ones_v, cnt_sh.at[pl.ds(off, CHUNK)])
    if with_cnt:
        def fill_ones(i, _):
            ones_v[i, pl.ds(0, 16)] = jnp.ones((16,), jnp.float32)
            return 0
        lax.fori_loop(0, CHUNK, fill_ones, 0)
    plsc.subcore_barrier()

    def drain_idx():
        pltpu.make_async_copy(src_hbm.at[pl.ds(0, CHUNK)], src4[0], isem).wait()
        pltpu.make_async_copy(src_hbm.at[pl.ds(0, CHUNK)], dst4[0], isem).wait()

    def drain_scat(b):
        for q in range(N_COL):
            pltpu.make_async_copy(h_hbms[q].at[pl.ds(0, CHUNK)],
                                  rows2[b][q], ssem[b]).wait()
        if with_cnt:
            pltpu.make_async_copy(cnt_hbm.at[pl.ds(0, CHUNK)],
                                  ones_v, ssem[b]).wait()

    pltpu.async_copy(src_hbm.at[pl.ds(edge_base, CHUNK)], src4[0], isem)
    pltpu.async_copy(dst_hbm.at[pl.ds(edge_base, CHUNK)], dst4[0], isem)

    @pl.loop(0, n_chunks // 4)
    def quad_step(p):
        for u in range(4):
            j = p * 4 + u
            rb = u % 2
            drain_idx()
            @pl.when(j >= 2)
            def _():
                drain_scat(rb)
            off_n = edge_base + lax.rem(j + 1, n_chunks) * CHUNK
            nu = (u + 1) % 4
            pltpu.async_copy(src_hbm.at[pl.ds(off_n, CHUNK)], src4[nu], isem)
            pltpu.async_copy(dst_hbm.at[pl.ds(off_n, CHUNK)], dst4[nu], isem)
            gathers = [pltpu.async_copy(h_hbms[q].at[src4[u]], rows2[rb][q],
                                        gsem[rb])
                       for q in range(N_COL)]
            for g in gathers:
                g.wait()
            for q in range(N_COL):
                pltpu.async_copy(rows2[rb][q], accs[q].at[dst4[u]], ssem[rb],
                                 add=True)
            if with_cnt:
                pltpu.async_copy(ones_v, cnt_sh.at[dst4[u]], ssem[rb],
                                 add=True)

    drain_scat(0)
    drain_scat(1)
    drain_idx()
    plsc.subcore_barrier()

    @pl.loop(0, rows_per_tile // CHUNK)
    def flush(k):
        off = base + k * CHUNK
        for q in range(N_COL):
            pltpu.sync_copy(accs[q].at[pl.ds(off, CHUNK)], rows_qs[q])
            pltpu.sync_copy(rows_qs[q], parts[q].at[pl.ds(c * rows + off, CHUNK)])
        if with_cnt:
            pltpu.sync_copy(cnt_sh.at[pl.ds(off, CHUNK)], ones_v)
            pltpu.sync_copy(ones_v, cnt_hbm.at[pl.ds(c * rows + off, CHUNK)])


def _make_sc_scatter(rows, n_chunks, with_cnt):
    mesh = plsc.VectorSubcoreMesh(core_axis_name="c", subcore_axis_name="s")
    out_type = [jax.ShapeDtypeStruct((N_CORES * rows, CW), jnp.float32)
                for _ in range(N_COL)]
    if with_cnt:
        out_type.append(jax.ShapeDtypeStruct((N_CORES * rows, CNT_W), jnp.float32))
    scratch = [pltpu.VMEM_SHARED((rows, CW), jnp.float32) for _ in range(N_COL)]
    if with_cnt:
        scratch.append(pltpu.VMEM_SHARED((rows, CNT_W), jnp.float32))
    scratch += [pltpu.VMEM((CHUNK, CW), jnp.float32)
                for _ in range(2 * N_COL)]
    scratch += [pltpu.VMEM((CHUNK,), jnp.int32)
                for _ in range(8)]
    if with_cnt:
        scratch.append(pltpu.VMEM((CHUNK, CNT_W), jnp.float32))
    scratch += [pltpu.SemaphoreType.DMA] * 5
    return pl.kernel(
        functools.partial(_sc_scatter_body, with_cnt),
        out_type=tuple(out_type),
        mesh=mesh,
        scratch_types=tuple(scratch),
        compiler_params=pltpu.CompilerParams(use_tc_tiling_on_sc=False),
    )


def _split_store(o_refs, val):
    for q in range(N_COL):
        o_refs[q][...] = val[:, q * CW:(q + 1) * CW]


def _enc_body(x_ref, w_ref, b_ref, *o_refs):
    _split_store(o_refs, jnp.dot(x_ref[...], w_ref[...],
                                 preferred_element_type=jnp.float32) + b_ref[...])


def _mean_h(p_refs, cnt_ref, h_refs):
    agg = jnp.concatenate([r[...][0] + r[...][1] for r in p_refs], axis=1)
    deg = cnt_ref[0, :, :1] + cnt_ref[1, :, :1]
    mean = agg * (1.0 / jnp.maximum(deg, 1.0))
    h = jnp.concatenate([r[...] for r in h_refs], axis=1)
    return mean, h


def _combine_body(*refs):
    p_refs, cnt_ref, h_refs = refs[:N_COL], refs[N_COL], refs[N_COL + 1:2 * N_COL + 1]
    wl_ref, wr_ref, b_ref = refs[2 * N_COL + 1:2 * N_COL + 4]
    o_refs = refs[2 * N_COL + 4:]
    mean, h = _mean_h(p_refs, cnt_ref, h_refs)
    o = (jnp.dot(mean, wl_ref[...], preferred_element_type=jnp.float32)
         + jnp.dot(h, wr_ref[...], preferred_element_type=jnp.float32)
         + b_ref[...])
    _split_store(o_refs, jnp.maximum(o, 0.0))


def _combine_head_body(*refs):
    p_refs, cnt_ref, h_refs = refs[:N_COL], refs[N_COL], refs[N_COL + 1:2 * N_COL + 1]
    (wl_ref, wr_ref, b_ref, wm1_ref, bm1_ref, wm2_ref, bm2_ref,
     o_ref) = refs[2 * N_COL + 1:]
    mean, h = _mean_h(p_refs, cnt_ref, h_refs)
    h2 = (jnp.dot(mean, wl_ref[...], preferred_element_type=jnp.float32)
          + jnp.dot(h, wr_ref[...], preferred_element_type=jnp.float32)
          + b_ref[...])
    h2 = jnp.maximum(h2, 0.0)
    r = jnp.maximum(jnp.dot(h2, wm1_ref[...],
                            preferred_element_type=jnp.float32) + bm1_ref[...],
                    0.0)
    logit = jnp.sum(r * wm2_ref[...], axis=1) + bm2_ref[0, 0]
    o_ref[...] = (1.0 / (1.0 + jnp.exp(-logit)))[None, :]


def _full_spec(shape):
    return pl.BlockSpec(shape, lambda i: tuple(0 for _ in shape))


def kernel(x, edge_index, edge_attr, batch, W_enc, b_enc, W1_l, b1_l, W1_r,
           W2_l, b2_l, W2_r, W_m1, b_m1, W_m2, b_m2):
    n = x.shape[0]
    e = edge_index.shape[1]
    rows = pl.cdiv(n + 1, BLK) * BLK
    n_chunks = pl.cdiv(pl.cdiv(e, NW * CHUNK), 4) * 4
    e_pad = NW * CHUNK * n_chunks

    src = edge_index[0].astype(jnp.int32)
    dst = edge_index[1].astype(jnp.int32)
    src = jnp.concatenate([src, jnp.zeros((e_pad - e,), jnp.int32)])
    dst = jnp.concatenate([dst, jnp.full((e_pad - e,), n, jnp.int32)])

    x_p = jnp.pad(x, ((0, rows - n), (0, 0)))
    b_enc2 = b_enc.reshape(1, D)
    b1 = b1_l.reshape(1, D)
    b2 = b2_l.reshape(1, D)
    bm1 = b_m1.reshape(1, D)
    wm2 = W_m2.reshape(1, D)
    bm2 = b_m2.reshape(1, 1)

    grid = rows // BLK
    row_spec = pl.BlockSpec((BLK, CW), lambda i: (i, 0))
    part_spec = pl.BlockSpec((N_CORES, BLK, CW), lambda i: (0, i, 0))
    cnt_spec = pl.BlockSpec((N_CORES, BLK, CNT_W), lambda i: (0, i, 0))
    nq_shape = [jax.ShapeDtypeStruct((rows, CW), jnp.float32)] * N_COL
    nq_spec = [row_spec] * N_COL

    h0s = pl.pallas_call(
        _enc_body,
        grid=(grid,),
        in_specs=[pl.BlockSpec((BLK, D), lambda i: (i, 0)),
                  _full_spec((D, D)), _full_spec((1, D))],
        out_specs=nq_spec,
        out_shape=nq_shape,
    )(x_p, W_enc, b_enc2)

    sc1 = _make_sc_scatter(rows, n_chunks, with_cnt=True)
    out1 = sc1(*h0s, src, dst)
    part1 = [p.reshape(N_CORES, rows, CW) for p in out1[:N_COL]]
    cnt = out1[N_COL].reshape(N_CORES, rows, CNT_W)

    h1s = pl.pallas_call(
        _combine_body,
        grid=(grid,),
        in_specs=[part_spec] * N_COL + [cnt_spec] + nq_spec
                 + [_full_spec((D, D)), _full_spec((D, D)), _full_spec((1, D))],
        out_specs=nq_spec,
        out_shape=nq_shape,
    )(*part1, cnt, *h0s, W1_l, W1_r, b1)

    sc2 = _make_sc_scatter(rows, n_chunks, with_cnt=False)
    out2 = sc2(*h1s, src, dst)
    part2 = [p.reshape(N_CORES, rows, CW) for p in out2]

    out2d = pl.pallas_call(
        _combine_head_body,
        grid=(grid,),
        in_specs=[part_spec] * N_COL + [cnt_spec] + nq_spec
                 + [_full_spec((D, D)), _full_spec((D, D)), _full_spec((1, D)),
                    _full_spec((D, D)), _full_spec((1, D)), _full_spec((1, D)),
                    _full_spec((1, 1))],
        out_specs=pl.BlockSpec((1, BLK), lambda i: (0, i)),
        out_shape=jax.ShapeDtypeStruct((1, rows), jnp.float32),
    )(*part2, cnt, *h1s, W2_l, W2_r, b2, W_m1, bm1, wm2, bm2)

    return out2d[0, :n]
